# baseline (speedup 1.0000x reference)
import functools

import jax
import jax.numpy as jnp
from jax import lax
from jax.experimental import pallas as pl
from jax.experimental.pallas import tpu as pltpu
from jax.experimental.pallas import tpu_sc as plsc

_FIELD = 100000
_NF = 26
_D = 16
_ROWS = 16384
_V = _NF * _FIELD
_CHUNK = 6400
_NFULL = _V // _CHUNK
_TAIL = _V - _NFULL * _CHUNK
_QL = 4096


@functools.cache
def _build():
    info = plsc.get_sparse_core_info()
    nc, ns = info.num_cores, info.num_subcores
    nw = nc * ns
    jobs_per_w = _NF * _D // nw
    max_ch = -(-_NFULL // nw)

    mesh = plsc.VectorSubcoreMesh(core_axis_name="c", subcore_axis_name="s")

    @functools.partial(
        pl.kernel,
        mesh=mesh,
        compiler_params=pltpu.CompilerParams(use_tc_tiling_on_sc=True),
        out_type=jax.ShapeDtypeStruct((_D, _V), jnp.float32),
        scratch_types=[pltpu.SemaphoreType.DMA],
    )
    def k1(tt_hbm, out_hbm, sem):
        wid = lax.axis_index("s") * nc + lax.axis_index("c")
        copies = []
        for i in range(max_ch):
            chunk = wid + nw * i
            if i == max_ch - 1:
                pred = chunk < _NFULL
            else:
                pred = None
            c0 = chunk * _CHUNK

            def fire(c0=c0):
                copies.append(
                    pltpu.async_copy(tt_hbm.at[:, pl.ds(c0, _CHUNK)],
                                     out_hbm.at[:, pl.ds(c0, _CHUNK)], sem))

            if pred is None:
                fire()
            else:
                @pl.when(pred)
                def _():
                    pltpu.async_copy(
                        tt_hbm.at[:, pl.ds(c0, _CHUNK)],
                        out_hbm.at[:, pl.ds(c0, _CHUNK)], sem).wait()

        @pl.when(wid == 0)
        def _():
            c0 = _NFULL * _CHUNK
            pltpu.async_copy(tt_hbm.at[:, pl.ds(c0, _TAIL)],
                             out_hbm.at[:, pl.ds(c0, _TAIL)], sem).wait()

        for cp in copies:
            cp.wait()

    @functools.partial(
        pl.kernel,
        mesh=mesh,
        compiler_params=pltpu.CompilerParams(
            use_tc_tiling_on_sc=False, needs_layout_passes=False),
        out_type=jax.ShapeDtypeStruct((_NF, 2, 128, 8, 128), jnp.float32),
        scratch_types=[
            pltpu.VMEM((_FIELD,), jnp.float32),
            pltpu.VMEM((_QL,), jnp.int32),
            pltpu.VMEM((2, _QL // 128, 128), jnp.float32),
            pltpu.SemaphoreType.DMA,
            pltpu.SemaphoreType.DMA,
            pltpu.SemaphoreType.DMA,
        ],
    )
    def k2(tlin_hbm, xt_hbm, out_hbm, band_v, idx_v, outc_v, bsem, isem, osem):
        wid = lax.axis_index("s") * nc + lax.axis_index("c")
        nbt = _QL // 128

        def do_job(j, carry):
            job = wid * jobs_per_w + j
            f = job // _D
            c = lax.rem(job, _D)
            cg = c // 8
            ch = lax.rem(c, 8)
            pltpu.async_copy(
                tlin_hbm.at[c, pl.ds(f * _FIELD, _FIELD)], band_v, bsem
            ).wait()

            def do_q(q, carry2):
                buf = lax.rem(q, 2)
                pltpu.sync_copy(xt_hbm.at[f, pl.ds(q * _QL, _QL)], idx_v)

                @pl.when(q > 1)
                def _():
                    pltpu.make_async_copy(
                        outc_v.at[buf],
                        out_hbm.at[0, 0, pl.ds(0, nbt), 0], osem).wait()

                def g(t, carry3):
                    for u in range(8):
                        i16 = idx_v[pl.ds(t * 128 + u * 16, 16)]
                        outc_v[buf, t, pl.ds(u * 16, 16)] = plsc.load_gather(
                            band_v, [i16])
                    return carry3

                lax.fori_loop(0, nbt, g, 0)
                pltpu.async_copy(
                    outc_v.at[buf],
                    out_hbm.at[f, cg, pl.ds(q * nbt, nbt), ch], osem)
                return carry2

            lax.fori_loop(0, _ROWS // _QL, do_q, 0)
            pltpu.make_async_copy(
                outc_v.at[0], out_hbm.at[0, 0, pl.ds(0, nbt), 0], osem).wait()
            pltpu.make_async_copy(
                outc_v.at[1], out_hbm.at[0, 0, pl.ds(0, nbt), 0], osem).wait()
            return carry

        lax.fori_loop(0, jobs_per_w, do_job, 0)

    return k1, k2


def kernel(x, table):
    k1, k2 = _build()
    tlin = k1(table.T)
    out5 = k2(tlin, x.T)
    return out5.transpose(2, 4, 0, 1, 3).reshape(_ROWS, _NF, _D)


# speedup vs baseline: 2.4396x; 2.4396x over previous
import functools

import jax
import jax.numpy as jnp
from jax import lax
from jax.experimental import pallas as pl
from jax.experimental.pallas import tpu as pltpu
from jax.experimental.pallas import tpu_sc as plsc

_FIELD = 100000
_NF = 26
_D = 16
_ROWS = 16384
_QL = 4096


@functools.cache
def _build():
    info = plsc.get_sparse_core_info()
    nc, ns = info.num_cores, info.num_subcores
    nw = nc * ns
    jobs_per_w = _NF * _D // nw

    mesh = plsc.VectorSubcoreMesh(core_axis_name="c", subcore_axis_name="s")

    @functools.partial(
        pl.kernel,
        mesh=mesh,
        compiler_params=pltpu.CompilerParams(
            use_tc_tiling_on_sc=False, needs_layout_passes=False),
        out_type=jax.ShapeDtypeStruct((_NF, 2, 128, 8, 128), jnp.float32),
        scratch_types=[
            pltpu.VMEM((_FIELD,), jnp.float32),
            pltpu.VMEM((_QL,), jnp.int32),
            pltpu.VMEM((2, _QL // 128, 128), jnp.float32),
            pltpu.SemaphoreType.DMA,
            pltpu.SemaphoreType.DMA,
        ],
    )
    def k2(tlin_hbm, xt_hbm, out_hbm, band_v, idx_v, outc_v, bsem, osem):
        wid = lax.axis_index("s") * nc + lax.axis_index("c")
        nbt = _QL // 128

        def do_job(j, carry):
            job = wid * jobs_per_w + j
            f = job // _D
            c = lax.rem(job, _D)
            cg = c // 8
            ch = lax.rem(c, 8)
            pltpu.async_copy(
                tlin_hbm.at[c, pl.ds(f * _FIELD, _FIELD)], band_v, bsem
            ).wait()

            def do_q(q, carry2):
                buf = lax.rem(q, 2)
                pltpu.sync_copy(xt_hbm.at[f, pl.ds(q * _QL, _QL)], idx_v)

                @pl.when(q > 1)
                def _():
                    pltpu.make_async_copy(
                        outc_v.at[buf],
                        out_hbm.at[0, 0, pl.ds(0, nbt), 0], osem).wait()

                def g(t, carry3):
                    for u in range(8):
                        i16 = idx_v[pl.ds(t * 128 + u * 16, 16)]
                        outc_v[buf, t, pl.ds(u * 16, 16)] = plsc.load_gather(
                            band_v, [i16])
                    return carry3

                lax.fori_loop(0, nbt, g, 0)
                pltpu.async_copy(
                    outc_v.at[buf],
                    out_hbm.at[f, cg, pl.ds(q * nbt, nbt), ch], osem)
                return carry2

            lax.fori_loop(0, _ROWS // _QL, do_q, 0)
            pltpu.make_async_copy(
                outc_v.at[0], out_hbm.at[0, 0, pl.ds(0, nbt), 0], osem).wait()
            pltpu.make_async_copy(
                outc_v.at[1], out_hbm.at[0, 0, pl.ds(0, nbt), 0], osem).wait()
            return carry

        lax.fori_loop(0, jobs_per_w, do_job, 0)

    return k2


def kernel(x, table):
    k2 = _build()
    out5 = k2(table.T, x.T)
    return out5.transpose(2, 4, 0, 1, 3).reshape(_ROWS, _NF, _D)


# speedup vs baseline: 6.6794x; 2.7379x over previous
import functools

import jax
import jax.numpy as jnp
from jax import lax
from jax.experimental import pallas as pl
from jax.experimental.pallas import tpu as pltpu
from jax.experimental.pallas import tpu_sc as plsc

_FIELD = 100000
_NF = 26
_D = 16
_ROWS = 16384
_BPW = 512
_CH = 256


@functools.cache
def _build():
    info = plsc.get_sparse_core_info()
    nc, ns = info.num_cores, info.num_subcores
    nw = nc * ns
    assert _ROWS == nw * _BPW
    nchunk = _BPW // _CH

    mesh = plsc.VectorSubcoreMesh(core_axis_name="c", subcore_axis_name="s")

    @functools.partial(
        pl.kernel,
        mesh=mesh,
        compiler_params=pltpu.CompilerParams(
            use_tc_tiling_on_sc=False, needs_layout_passes=False),
        out_type=jax.ShapeDtypeStruct((_NF, 2, 128, 8, 128), jnp.float32),
        scratch_types=[
            pltpu.VMEM((_NF, _BPW), jnp.int32),
            pltpu.VMEM((2, _CH), jnp.int32),
            pltpu.VMEM((2, _CH), jnp.int32),
            pltpu.VMEM((2, _CH, 128), jnp.float32),
            pltpu.VMEM((2, 2, 2, 8, 128), jnp.float32),
            pltpu.SemaphoreType.DMA,
            pltpu.SemaphoreType.DMA,
            pltpu.SemaphoreType.DMA,
        ],
    )
    def k(t8_hbm, xt_hbm, out_hbm, xv, srv, sbv, rowsv, outv,
          gsem0, gsem1, osem):
        wid = lax.axis_index("s") * nc + lax.axis_index("c")
        b0 = wid * _BPW
        btg0 = wid * (_BPW // 128)
        pltpu.sync_copy(xt_hbm.at[:, pl.ds(b0, _BPW)], xv)
        lane = lax.iota(jnp.int32, 16)

        def prep(k_, carry):
            f = k_ // nchunk
            h = lax.rem(k_, nchunk)
            buf = lax.rem(k_, 2)

            def add(g, carry2):
                s = pl.ds(g * 16, 16)
                adj = xv[f, pl.ds(h * _CH + g * 16, 16)] + f * _FIELD
                srv[buf, s] = lax.shift_right_logical(adj, 3)
                sbv[buf, s] = lax.shift_left(
                    lax.bitwise_and(adj, 7), 4)
                return carry2

            lax.fori_loop(0, _CH // 16, add, 0)
            return carry

        def gather_start(k_):
            buf = lax.rem(k_, 2)

            @pl.when(lax.rem(k_, 2) == 0)
            def _():
                pltpu.async_copy(t8_hbm.at[srv.at[0]], rowsv.at[0], gsem0)

            @pl.when(lax.rem(k_, 2) == 1)
            def _():
                pltpu.async_copy(t8_hbm.at[srv.at[1]], rowsv.at[1], gsem1)

        def gather_wait(k_):
            @pl.when(lax.rem(k_, 2) == 0)
            def _():
                pltpu.make_async_copy(
                    t8_hbm.at[srv.at[0]], rowsv.at[0], gsem0).wait()

            @pl.when(lax.rem(k_, 2) == 1)
            def _():
                pltpu.make_async_copy(
                    t8_hbm.at[srv.at[1]], rowsv.at[1], gsem1).wait()

        prep(0, 0)
        gather_start(0)

        def do_chunk(k_, carry):
            nk = k_ + 1
            buf = lax.rem(k_, 2)

            @pl.when(nk < _NF * nchunk)
            def _():
                prep(nk, 0)
                gather_start(nk)

            gather_wait(k_)
            f = k_ // nchunk
            h = lax.rem(k_, nchunk)

            @pl.when(k_ > 1)
            def _():
                pltpu.make_async_copy(
                    outv.at[buf], out_hbm.at[0, :, pl.ds(0, 2)], osem).wait()

            def trans(t, carry3):
                base = t * 16
                ridx = lane + base
                cidx0 = sbv[buf, pl.ds(base, 16)]
                bt = t // 8
                bg = lax.rem(t, 8)
                for cg in range(2):
                    for ch in range(8):
                        outv[buf, cg, bt, ch, pl.ds(bg * 16, 16)] = (
                            plsc.load_gather(
                                rowsv.at[buf], [ridx, cidx0 + (cg * 8 + ch)]))
                return carry3

            lax.fori_loop(0, _CH // 16, trans, 0)
            pltpu.async_copy(
                outv.at[buf],
                out_hbm.at[f, :, pl.ds(btg0 + h * 2, 2)], osem)
            return carry

        lax.fori_loop(0, _NF * nchunk, do_chunk, 0)
        pltpu.make_async_copy(
            outv.at[0], out_hbm.at[0, :, pl.ds(0, 2)], osem).wait()
        pltpu.make_async_copy(
            outv.at[1], out_hbm.at[0, :, pl.ds(0, 2)], osem).wait()

    return k


def kernel(x, table):
    k = _build()
    out5 = k(table.reshape(_FIELD * _NF // 8, 128), x.T)
    return out5.transpose(2, 4, 0, 1, 3).reshape(_ROWS, _NF, _D)
